# baseline (speedup 1.0000x reference)
import functools

import jax
import jax.numpy as jnp
from jax import lax
from jax.experimental import pallas as pl
from jax.experimental.pallas import tpu as pltpu
from jax.experimental.pallas import tpu_sc as plsc

_N = 100000
_D = 128
_BLK = 1000

_NW = 32
_NB = 8
_ROWS = 12544
_NPAD = _NB * _ROWS
_DEGW = 101376
_EP = 100352
_EWK = _EP // _NW
_SLOT = _EWK + 64
_FIRE = 64
_SENT = 0x3FFFFFFF
_SHIFT = 17


def _mesh():
    return plsc.VectorSubcoreMesh(core_axis_name="c", subcore_axis_name="s")


def _bin_body(src_ref, dst_ref, bins_ref, cnts_ref, deg_ref,
              esrc, edst, ebuf, didx, ones64, cntv, zbuf, deg_sh):
    c = lax.axis_index("c")
    s = lax.axis_index("s")
    w = c * 16 + s
    iota16 = lax.iota(jnp.int32, 16)
    zero16 = jnp.zeros((16,), jnp.float32)
    one16 = jnp.ones((16,), jnp.float32)

    pltpu.sync_copy(src_ref.at[pl.ds(w * _EWK, _EWK)], esrc)
    pltpu.sync_copy(dst_ref.at[pl.ds(w * _EWK, _EWK)], edst)

    def _zb(r, carry):
        zbuf[pl.ds(r * 16, 16)] = zero16
        return carry
    lax.fori_loop(0, 198, _zb, 0)
    for t in range(4):
        ones64[pl.ds(t * 16, 16)] = one16
    cntv[pl.ds(0, 16)] = jnp.zeros((16,), jnp.int32)

    pltpu.sync_copy(zbuf, deg_sh.at[pl.ds(s * 6336, 3168)])
    pltpu.sync_copy(zbuf, deg_sh.at[pl.ds(s * 6336 + 3168, 3168)])
    plsc.subcore_barrier()

    for blk in range(_NB):
        lo = blk * _ROWS

        def _scan(k, cnt):
            base = k * 16
            dvec = edst[pl.ds(base, 16)]
            svec = esrc[pl.ds(base, 16)]
            rel = dvec - lo
            m = (rel >= 0) & (rel < _ROWS)
            mi = m.astype(jnp.int32)
            pos = cnt + plsc.cumsum(mi) - 1
            pk = svec | lax.shift_left(rel, _SHIFT)
            plsc.store_scatter(ebuf, [pos], pk, mask=m)
            return cnt + jnp.sum(mi)
        cnt = lax.fori_loop(0, _EWK // 16, _scan, jnp.int32(0))

        pad = jnp.full((16,), _ROWS << _SHIFT, jnp.int32)
        for t in range(4):
            plsc.store_scatter(ebuf, [cnt + t * 16 + iota16], pad)
        nf = (cnt + (_FIRE - 1)) // _FIRE
        plsc.store_scatter(cntv, [jnp.full((16,), blk, jnp.int32)],
                           jnp.full((16,), nf, jnp.int32), mask=iota16 == 0)

        def _degf(f, carry):
            for t in range(4):
                pk = ebuf[pl.ds(f * _FIRE + t * 16, 16)]
                rel = lax.shift_right_logical(pk, _SHIFT)
                didx[pl.ds(t * 16, 16)] = jnp.where(rel >= _ROWS, _NPAD, rel + lo)
            pltpu.sync_copy(ones64, deg_sh.at[didx], add=True)
            return carry
        lax.fori_loop(0, nf, _degf, 0)

        pltpu.sync_copy(ebuf, bins_ref.at[pl.ds((blk * _NW + w) * _SLOT, _SLOT)])

    pltpu.sync_copy(cntv, cnts_ref.at[pl.ds(w * 16, 16)])
    plsc.subcore_barrier()
    for h in range(2):
        pltpu.sync_copy(deg_sh.at[pl.ds(s * 6336 + h * 3168, 3168)], zbuf)
        pltpu.sync_copy(zbuf, deg_ref.at[pl.ds(c * _DEGW + s * 6336 + h * 3168, 3168)])


def _bin(src, dst):
    kern = pl.kernel(
        _bin_body,
        out_type=(
            jax.ShapeDtypeStruct((_NB * _NW * _SLOT,), jnp.int32),
            jax.ShapeDtypeStruct((_NW * 16,), jnp.int32),
            jax.ShapeDtypeStruct((2 * _DEGW,), jnp.float32),
        ),
        mesh=_mesh(),
        compiler_params=pltpu.CompilerParams(needs_layout_passes=False),
        scratch_types=[
            pltpu.VMEM((_EWK,), jnp.int32),
            pltpu.VMEM((_EWK,), jnp.int32),
            pltpu.VMEM((_SLOT,), jnp.int32),
            pltpu.VMEM((_FIRE,), jnp.int32),
            pltpu.VMEM((_FIRE,), jnp.float32),
            pltpu.VMEM((16,), jnp.int32),
            pltpu.VMEM((3168,), jnp.float32),
            pltpu.VMEM_SHARED((_DEGW,), jnp.float32),
        ],
    )
    return kern(src, dst)


_F3 = 32
_CB = 4096
_FR = _ROWS // 16


def _smooth_body(width, g_ref, bins_ref, cnts_ref, s_ref,
                 cntb, sbuf, cbuf,
                 g0, g1, g2, g3, x0, x1, x2, x3, r0, r1, r2, r3,
                 acc, semg):
    c = lax.axis_index("c")
    s = lax.axis_index("s")
    iota16 = lax.iota(jnp.int32, 16)
    zero16 = jnp.zeros((16,), jnp.float32)
    mask17 = (1 << _SHIFT) - 1
    sh = s * _FR
    pk_shift = lax.shift_left(sh, _SHIFT)
    slots = [(g0, x0, r0), (g1, x1, r1), (g2, x2, r2), (g3, x3, r3)]

    pltpu.sync_copy(cnts_ref, cntb)

    def _prep(f, gk, xk):
        for t in range(2):
            pk = cbuf[pl.ds(f * _F3 + t * 16, 16)]
            gk[pl.ds(t * 16, 16)] = pk & mask17
            xk[pl.ds(t * 16, 16)] = lax.shift_right_logical(pk, _SHIFT)

    def _accum(rk, xk):
        rv0 = iota16
        rv1 = iota16 + 16
        fr0 = xk[pl.ds(0, 16)]
        fr1 = xk[pl.ds(16, 16)]

        def dbody(d, carry):
            dv = jnp.full((16,), 0, jnp.int32) + d
            v0 = plsc.load_gather(rk, [rv0, dv])
            plsc.addupdate_scatter(acc, [fr0, dv], v0)
            v1 = plsc.load_gather(rk, [rv1, dv])
            plsc.addupdate_scatter(acc, [fr1, dv], v1)
            return carry
        lax.fori_loop(0, width, dbody, 0)

    def _fires(nf):
        for k in range(3):
            gk, xk, rk = slots[k]

            @pl.when(k < nf)
            def _(gk=gk, xk=xk, rk=rk, k=k):
                _prep(k, gk, xk)
                pltpu.async_copy(g_ref.at[gk], rk, semg)

        def _body(q, carry):
            for k in range(4):
                f = q * 4 + k
                gk, xk, rk = slots[k]
                gn, xn, rn = slots[(k + 3) % 4]

                @pl.when(f < nf)
                def _(f=f, gk=gk, xk=xk, rk=rk, gn=gn, xn=xn, rn=rn):
                    pltpu.make_async_copy(g_ref.at[gk], rk, semg).wait()

                    @pl.when(f + 3 < nf)
                    def _():
                        _prep(f + 3, gn, xn)
                        pltpu.async_copy(g_ref.at[gn], rn, semg)
                    _accum(rk, xk)
            return carry
        lax.fori_loop(0, (nf + 3) // 4, _body, 0)

    for t in range(_NB // 2):
        b = c * (_NB // 2) + t
        lo = b * _ROWS

        def _zb(r, carry):
            for t2 in range(width // 16):
                acc[r, pl.ds(t2 * 16, 16)] = zero16
            return carry
        lax.fori_loop(0, _FR, _zb, 0)

        def _slot(w2, cnt):
            cvec = cntb[pl.ds(w2 * 16, 16)]
            nf_s = jnp.sum(jnp.where(iota16 == b, cvec, 0))
            nwords = nf_s * _FIRE
            nfull = jnp.where(cnt + nwords > _CB - _F3, cnt // _F3, 0)
            _fires(nfull)
            for t2 in range(2):
                v = cbuf[pl.ds(nfull * _F3 + t2 * 16, 16)]
                cbuf[pl.ds(t2 * 16, 16)] = v
            cnt = cnt - nfull * _F3

            slotbase = (b * _NW + w2) * _SLOT

            def _chunk(q, cnt):
                pltpu.sync_copy(
                    bins_ref.at[pl.ds(slotbase + q * 512, 512)], sbuf)
                steps = jnp.minimum(32, (nwords - q * 512) // 16)

                def _sc(k2, cnt):
                    pk = sbuf[pl.ds(k2 * 16, 16)]
                    frel = lax.shift_right_logical(pk, _SHIFT) - sh
                    m = (frel >= 0) & (frel < _FR)
                    mi = m.astype(jnp.int32)
                    pos = cnt + plsc.cumsum(mi) - 1
                    plsc.store_scatter(cbuf, [pos], pk - pk_shift, mask=m)
                    return cnt + jnp.sum(mi)
                return lax.fori_loop(0, steps, _sc, cnt)
            return lax.fori_loop(0, (nwords + 511) // 512, _chunk, cnt)
        cnt = lax.fori_loop(0, _NW, _slot, jnp.int32(0))

        pad = jnp.full((16,), _FR << _SHIFT, jnp.int32)
        for t2 in range(2):
            plsc.store_scatter(cbuf, [cnt + t2 * 16 + iota16], pad)
        _fires((cnt + _F3 - 1) // _F3)

        pltpu.sync_copy(acc.at[pl.ds(0, _FR)],
                        s_ref.at[pl.ds(lo + sh, _FR)])


def _smooth(G, bins, cnts):
    width = G.shape[1]
    kern = pl.kernel(
        functools.partial(_smooth_body, width),
        out_type=jax.ShapeDtypeStruct((_NPAD, width), jnp.float32),
        mesh=_mesh(),
        compiler_params=pltpu.CompilerParams(needs_layout_passes=False),
        scratch_types=[
            pltpu.VMEM((_NW * 16,), jnp.int32),
            pltpu.VMEM((512,), jnp.int32),
            pltpu.VMEM((_CB,), jnp.int32),
            pltpu.VMEM((_F3,), jnp.int32),
            pltpu.VMEM((_F3,), jnp.int32),
            pltpu.VMEM((_F3,), jnp.int32),
            pltpu.VMEM((_F3,), jnp.int32),
            pltpu.VMEM((_F3,), jnp.int32),
            pltpu.VMEM((_F3,), jnp.int32),
            pltpu.VMEM((_F3,), jnp.int32),
            pltpu.VMEM((_F3,), jnp.int32),
            pltpu.VMEM((_F3, width), jnp.float32),
            pltpu.VMEM((_F3, width), jnp.float32),
            pltpu.VMEM((_F3, width), jnp.float32),
            pltpu.VMEM((_F3, width), jnp.float32),
            pltpu.VMEM((_FR + 8, width), jnp.float32),
            pltpu.SemaphoreType.DMA,
        ],
    )
    return kern(G, bins, cnts)


def _l0_body(x_ref, w_ref, b_ref, deg_ref, o_ref):
    dinv = jax.lax.rsqrt(deg_ref[...])
    h = jnp.dot(x_ref[...], w_ref[...], preferred_element_type=jnp.float32)
    o_ref[...] = (h + b_ref[...]) * dinv


def _mid_body(s_ref, g_ref, deg_ref, w_ref, b_ref, o_ref):
    dinv = jax.lax.rsqrt(deg_ref[...])
    h_in = jnp.maximum(dinv * (s_ref[...] + g_ref[...]), 0.0)
    h = jnp.dot(h_in, w_ref[...], preferred_element_type=jnp.float32)
    o_ref[...] = (h + b_ref[...]) * dinv


def _fin_body(s_ref, g_ref, deg_ref, o_ref):
    dinv = jax.lax.rsqrt(deg_ref[...])
    o_ref[...] = dinv * (s_ref[...] + g_ref[...])


def _row_spec(width):
    return pl.BlockSpec((_BLK, width), lambda i: (i, 0))


def _full_spec(shape):
    return pl.BlockSpec(shape, lambda i: (0, 0))


def _layer0(X, W, b, deg):
    return pl.pallas_call(
        _l0_body,
        grid=(_N // _BLK,),
        in_specs=[
            _row_spec(_D),
            _full_spec(W.shape),
            _full_spec((1, W.shape[1])),
            _row_spec(1),
        ],
        out_specs=_row_spec(W.shape[1]),
        out_shape=jax.ShapeDtypeStruct((_N, W.shape[1]), jnp.float32),
    )(X, W, b.reshape(1, -1), deg)


def _layer_mid(S, G, deg, W, b):
    return pl.pallas_call(
        _mid_body,
        grid=(_N // _BLK,),
        in_specs=[
            _row_spec(_D),
            _row_spec(_D),
            _row_spec(1),
            _full_spec(W.shape),
            _full_spec((1, W.shape[1])),
        ],
        out_specs=_row_spec(W.shape[1]),
        out_shape=jax.ShapeDtypeStruct((_N, W.shape[1]), jnp.float32),
    )(S, G, deg, W, b.reshape(1, -1))


def _layer_fin(S, G, deg):
    width = G.shape[1]
    return pl.pallas_call(
        _fin_body,
        grid=(_N // _BLK,),
        in_specs=[_row_spec(width), _row_spec(width), _row_spec(1)],
        out_specs=_row_spec(width),
        out_shape=jax.ShapeDtypeStruct((_N, width), jnp.float32),
    )(S, G, deg)


def kernel(X, hyperedges, W0, b0, W1, b1, W2, b2):
    he = hyperedges.astype(jnp.int32)
    E, K = he.shape

    Xe = X[he]
    sq = jnp.sum(Xe * Xe, axis=-1)
    gram = jnp.einsum('ekd,emd->ekm', Xe, Xe)
    dist = sq[:, :, None] + sq[:, None, :] - 2.0 * gram
    flat = jnp.argmax(dist.reshape(E, K * K), axis=1)
    i = flat // K
    j = flat % K
    ar = jnp.arange(E)
    u = he[ar, i]
    v = he[ar, j]
    src = jnp.concatenate([u, v])
    dst = jnp.concatenate([v, u])

    npad = _EP - src.shape[0]
    src_p = jnp.concatenate([src, jnp.zeros((npad,), jnp.int32)])
    dst_p = jnp.concatenate([dst, jnp.full((npad,), _SENT, jnp.int32)])

    bins, cnts, degp = _bin(src_p, dst_p)
    deg = (degp[:_N] + degp[_DEGW:_DEGW + _N] + 1.0).reshape(_N, 1)

    W2p = jnp.pad(W2, ((0, 0), (0, 88)))
    b2p = jnp.pad(b2, (0, 88))

    G0 = _layer0(X, W0, b0, deg)
    S0 = _smooth(G0, bins, cnts)[:_N]
    G1 = _layer_mid(S0, G0, deg, W1, b1)
    S1 = _smooth(G1, bins, cnts)[:_N]
    G2 = _layer_mid(S1, G1, deg, W2p, b2p)
    S2 = _smooth(G2, bins, cnts)[:_N]
    return _layer_fin(S2, G2, deg)[:, :40]


# speedup vs baseline: 1.0390x; 1.0390x over previous
import functools

import jax
import jax.numpy as jnp
from jax import lax
from jax.experimental import pallas as pl
from jax.experimental.pallas import tpu as pltpu
from jax.experimental.pallas import tpu_sc as plsc

_N = 100000
_D = 128
_BLK = 1000

_NW = 32
_NB = 8
_ROWS = 12544
_NPAD = _NB * _ROWS
_DEGW = 101376
_EP = 100352
_EWK = _EP // _NW
_SLOT = _EWK + 64
_FIRE = 64
_SENT = 0x3FFFFFFF
_SHIFT = 17


def _mesh():
    return plsc.VectorSubcoreMesh(core_axis_name="c", subcore_axis_name="s")


_SEG = 512
_HALFC = 16 * 3584
_BLKCAP = 2 * _HALFC
_EBCAP = _EWK + _SEG + 64


def _bin_body(src_ref, dst_ref, bins_ref, cnts_ref, deg_ref,
              esrc, edst, ebuf, didx, ones64, cntv, zbuf, cnt_sh, deg_sh):
    c = lax.axis_index("c")
    s = lax.axis_index("s")
    w = c * 16 + s
    iota16 = lax.iota(jnp.int32, 16)
    zero16 = jnp.zeros((16,), jnp.float32)
    one16 = jnp.ones((16,), jnp.float32)

    pltpu.sync_copy(src_ref.at[pl.ds(w * _EWK, _EWK)], esrc)
    pltpu.sync_copy(dst_ref.at[pl.ds(w * _EWK, _EWK)], edst)

    def _zb(r, carry):
        zbuf[pl.ds(r * 16, 16)] = zero16
        return carry
    lax.fori_loop(0, 198, _zb, 0)
    for t in range(4):
        ones64[pl.ds(t * 16, 16)] = one16

    pltpu.sync_copy(zbuf, deg_sh.at[pl.ds(s * 6336, 3168)])
    pltpu.sync_copy(zbuf, deg_sh.at[pl.ds(s * 6336 + 3168, 3168)])

    def _cnt(k, carry):
        dvec = edst[pl.ds(k * 16, 16)]
        outs = []
        for b in range(_NB):
            rel = dvec - b * _ROWS
            m = (rel >= 0) & (rel < _ROWS)
            outs.append(carry[b] + jnp.sum(m.astype(jnp.int32)))
        return tuple(outs)
    cnts8 = lax.fori_loop(0, _EWK // 16, _cnt,
                          tuple(jnp.int32(0) for _ in range(_NB)))
    nch8 = [(cnt + (_SEG - 1)) // _SEG for cnt in cnts8]

    nchv = jnp.zeros((16,), jnp.int32)
    for b in range(_NB):
        nchv = jnp.where(iota16 == b, nch8[b], nchv)
    cntv[pl.ds(0, 16)] = nchv
    pltpu.sync_copy(cntv, cnt_sh.at[pl.ds(s * 16, 16)])
    plsc.subcore_barrier()
    pltpu.sync_copy(cnt_sh, ebuf.at[pl.ds(0, 256)])
    pref = jnp.zeros((16,), jnp.int32)
    tot = jnp.zeros((16,), jnp.int32)
    for s2 in range(16):
        v = ebuf[pl.ds(s2 * 16, 16)]
        pref = pref + jnp.where(jnp.full((16,), s2, jnp.int32) < s, v, 0)
        tot = tot + v

    @pl.when(s == 0)
    def _():
        cntv[pl.ds(0, 16)] = tot
        pltpu.sync_copy(cntv, cnts_ref.at[pl.ds(c * 16, 16)])

    pad = jnp.full((16,), _ROWS << _SHIFT, jnp.int32)
    for blk in range(_NB):
        lo = blk * _ROWS

        def _scan(k, cnt):
            base = k * 16
            dvec = edst[pl.ds(base, 16)]
            svec = esrc[pl.ds(base, 16)]
            rel = dvec - lo
            m = (rel >= 0) & (rel < _ROWS)
            mi = m.astype(jnp.int32)
            pos = cnt + plsc.cumsum(mi) - 1
            pk = svec | lax.shift_left(rel, _SHIFT)
            plsc.store_scatter(ebuf, [pos], pk, mask=m)
            return cnt + jnp.sum(mi)
        cnt = lax.fori_loop(0, _EWK // 16, _scan, jnp.int32(0))

        for t in range(_SEG // 16):
            plsc.store_scatter(ebuf, [cnt + t * 16 + iota16], pad)
        nch_b = (cnt + (_SEG - 1)) // _SEG
        pref_b = jnp.sum(jnp.where(iota16 == blk, pref, 0))
        base_w = blk * _BLKCAP + c * _HALFC + pref_b * _SEG

        def _degf(f, carry):
            for t in range(4):
                pk = ebuf[pl.ds(f * 64 + t * 16, 16)]
                rel = lax.shift_right_logical(pk, _SHIFT)
                didx[pl.ds(t * 16, 16)] = jnp.where(rel >= _ROWS, _NPAD, rel + lo)
            pltpu.sync_copy(ones64, deg_sh.at[didx], add=True)
            return carry
        lax.fori_loop(0, nch_b * (_SEG // 64), _degf, 0)

        def _wr(q, carry):
            pltpu.sync_copy(ebuf.at[pl.ds(q * _SEG, _SEG)],
                            bins_ref.at[pl.ds(base_w + q * _SEG, _SEG)])
            return carry
        lax.fori_loop(0, nch_b, _wr, 0)

    plsc.subcore_barrier()
    for h in range(2):
        pltpu.sync_copy(deg_sh.at[pl.ds(s * 6336 + h * 3168, 3168)], zbuf)
        pltpu.sync_copy(zbuf, deg_ref.at[pl.ds(c * _DEGW + s * 6336 + h * 3168, 3168)])


def _bin(src, dst):
    kern = pl.kernel(
        _bin_body,
        out_type=(
            jax.ShapeDtypeStruct((_NB * _BLKCAP + 2048,), jnp.int32),
            jax.ShapeDtypeStruct((32,), jnp.int32),
            jax.ShapeDtypeStruct((2 * _DEGW,), jnp.float32),
        ),
        mesh=_mesh(),
        compiler_params=pltpu.CompilerParams(needs_layout_passes=False),
        scratch_types=[
            pltpu.VMEM((_EWK,), jnp.int32),
            pltpu.VMEM((_EWK,), jnp.int32),
            pltpu.VMEM((_EBCAP,), jnp.int32),
            pltpu.VMEM((64,), jnp.int32),
            pltpu.VMEM((64,), jnp.float32),
            pltpu.VMEM((16,), jnp.int32),
            pltpu.VMEM((3168,), jnp.float32),
            pltpu.VMEM_SHARED((256,), jnp.int32),
            pltpu.VMEM_SHARED((_DEGW,), jnp.float32),
        ],
    )
    return kern(src, dst)


_F3 = 32
_CB = 4096
_FR = _ROWS // 16


def _smooth_body(width, g_ref, bins_ref, cnts_ref, s_ref,
                 cntb, sb0, sb1, cbuf,
                 g0, g1, g2, g3, x0, x1, x2, x3, r0, r1, r2, r3,
                 acc, semg, semc):
    c = lax.axis_index("c")
    s = lax.axis_index("s")
    iota16 = lax.iota(jnp.int32, 16)
    zero16 = jnp.zeros((16,), jnp.float32)
    mask17 = (1 << _SHIFT) - 1
    sh = s * _FR
    pk_shift = lax.shift_left(sh, _SHIFT)
    slots = [(g0, x0, r0), (g1, x1, r1), (g2, x2, r2), (g3, x3, r3)]
    _CH = 2048

    pltpu.sync_copy(cnts_ref, cntb)

    def _prep(f, gk, xk):
        for t in range(2):
            pk = cbuf[pl.ds(f * _F3 + t * 16, 16)]
            gk[pl.ds(t * 16, 16)] = pk & mask17
            xk[pl.ds(t * 16, 16)] = lax.shift_right_logical(pk, _SHIFT)

    def _accum(rk, xk):
        rv0 = iota16
        rv1 = iota16 + 16
        fr0 = xk[pl.ds(0, 16)]
        fr1 = xk[pl.ds(16, 16)]

        def dbody(d, carry):
            dv = jnp.full((16,), 0, jnp.int32) + d
            v0 = plsc.load_gather(rk, [rv0, dv])
            plsc.addupdate_scatter(acc, [fr0, dv], v0)
            v1 = plsc.load_gather(rk, [rv1, dv])
            plsc.addupdate_scatter(acc, [fr1, dv], v1)
            return carry
        lax.fori_loop(0, width, dbody, 0)

    def _fires(nf):
        for k in range(3):
            gk, xk, rk = slots[k]

            @pl.when(k < nf)
            def _(gk=gk, xk=xk, rk=rk, k=k):
                _prep(k, gk, xk)
                pltpu.async_copy(g_ref.at[gk], rk, semg)

        def _body(q, carry):
            for k in range(4):
                f = q * 4 + k
                gk, xk, rk = slots[k]
                gn, xn, rn = slots[(k + 3) % 4]

                @pl.when(f < nf)
                def _(f=f, gk=gk, xk=xk, rk=rk, gn=gn, xn=xn, rn=rn):
                    pltpu.make_async_copy(g_ref.at[gk], rk, semg).wait()

                    @pl.when(f + 3 < nf)
                    def _():
                        _prep(f + 3, gn, xn)
                        pltpu.async_copy(g_ref.at[gn], rn, semg)
                    _accum(rk, xk)
            return carry
        lax.fori_loop(0, (nf + 3) // 4, _body, 0)

    for t in range(_NB // 2):
        b = c * (_NB // 2) + t
        lo = b * _ROWS

        def _zb(r, carry):
            for t2 in range(width // 16):
                acc[r, pl.ds(t2 * 16, 16)] = zero16
            return carry
        lax.fori_loop(0, _FR, _zb, 0)

        cnt = jnp.int32(0)
        for h in range(2):
            base_w = b * _BLKCAP + h * _HALFC
            nch_tot = jnp.sum(jnp.where(iota16 == b, cntb[pl.ds(h * 16, 16)], 0))
            nwords = nch_tot * _SEG
            n2 = (nwords + _CH - 1) // _CH

            def _ld(q, sb):
                return pltpu.async_copy(
                    bins_ref.at[pl.ds(base_w + q * _CH, _CH)], sb, semc)

            def _scan_chunk(sb, q, cnt):
                steps = jnp.clip((nwords - q * _CH) // 16, 0, _CH // 16)

                def _sc(k2, cnt):
                    pk = sb[pl.ds(k2 * 16, 16)]
                    frel = lax.shift_right_logical(pk, _SHIFT) - sh
                    m = (frel >= 0) & (frel < _FR)
                    mi = m.astype(jnp.int32)
                    pos = cnt + plsc.cumsum(mi) - 1
                    plsc.store_scatter(cbuf, [pos], pk - pk_shift, mask=m)
                    return cnt + jnp.sum(mi)
                return lax.fori_loop(0, steps, _sc, cnt)

            @pl.when(n2 > 0)
            def _():
                _ld(0, sb0)

            def _cbody(qq, cnt):
                q0 = qq * 2
                q1 = q0 + 1
                nfull = jnp.where(cnt + 2 * _CH > _CB - _F3, cnt // _F3, 0)
                _fires(nfull)
                for t2 in range(2):
                    v = cbuf[pl.ds(nfull * _F3 + t2 * 16, 16)]
                    cbuf[pl.ds(t2 * 16, 16)] = v
                cnt = cnt - nfull * _F3

                pltpu.make_async_copy(
                    bins_ref.at[pl.ds(base_w + q0 * _CH, _CH)], sb0, semc).wait()

                @pl.when(q1 < n2)
                def _(q1=q1):
                    _ld(q1, sb1)
                cnt = _scan_chunk(sb0, q0, cnt)

                @pl.when(q1 < n2)
                def _(q1=q1):
                    pltpu.make_async_copy(
                        bins_ref.at[pl.ds(base_w + q1 * _CH, _CH)], sb1,
                        semc).wait()

                    @pl.when(q1 + 1 < n2)
                    def _():
                        _ld(q1 + 1, sb0)
                cnt = _scan_chunk(sb1, q1, cnt)
                return cnt
            cnt = lax.fori_loop(0, (n2 + 1) // 2, _cbody, cnt)

        pad = jnp.full((16,), _FR << _SHIFT, jnp.int32)
        for t2 in range(2):
            plsc.store_scatter(cbuf, [cnt + t2 * 16 + iota16], pad)
        _fires((cnt + _F3 - 1) // _F3)

        pltpu.sync_copy(acc.at[pl.ds(0, _FR)],
                        s_ref.at[pl.ds(lo + sh, _FR)])


def _smooth(G, bins, cnts):
    width = G.shape[1]
    kern = pl.kernel(
        functools.partial(_smooth_body, width),
        out_type=jax.ShapeDtypeStruct((_NPAD, width), jnp.float32),
        mesh=_mesh(),
        compiler_params=pltpu.CompilerParams(needs_layout_passes=False),
        scratch_types=[
            pltpu.VMEM((32,), jnp.int32),
            pltpu.VMEM((2048,), jnp.int32),
            pltpu.VMEM((2048,), jnp.int32),
            pltpu.VMEM((_CB,), jnp.int32),
            pltpu.VMEM((_F3,), jnp.int32),
            pltpu.VMEM((_F3,), jnp.int32),
            pltpu.VMEM((_F3,), jnp.int32),
            pltpu.VMEM((_F3,), jnp.int32),
            pltpu.VMEM((_F3,), jnp.int32),
            pltpu.VMEM((_F3,), jnp.int32),
            pltpu.VMEM((_F3,), jnp.int32),
            pltpu.VMEM((_F3,), jnp.int32),
            pltpu.VMEM((_F3, width), jnp.float32),
            pltpu.VMEM((_F3, width), jnp.float32),
            pltpu.VMEM((_F3, width), jnp.float32),
            pltpu.VMEM((_F3, width), jnp.float32),
            pltpu.VMEM((_FR + 8, width), jnp.float32),
            pltpu.SemaphoreType.DMA,
            pltpu.SemaphoreType.DMA,
        ],
    )
    return kern(G, bins, cnts)


def _l0_body(x_ref, w_ref, b_ref, deg_ref, o_ref):
    dinv = jax.lax.rsqrt(deg_ref[...])
    h = jnp.dot(x_ref[...], w_ref[...], preferred_element_type=jnp.float32)
    o_ref[...] = (h + b_ref[...]) * dinv


def _mid_body(s_ref, g_ref, deg_ref, w_ref, b_ref, o_ref):
    dinv = jax.lax.rsqrt(deg_ref[...])
    h_in = jnp.maximum(dinv * (s_ref[...] + g_ref[...]), 0.0)
    h = jnp.dot(h_in, w_ref[...], preferred_element_type=jnp.float32)
    o_ref[...] = (h + b_ref[...]) * dinv


def _fin_body(s_ref, g_ref, deg_ref, o_ref):
    dinv = jax.lax.rsqrt(deg_ref[...])
    o_ref[...] = dinv * (s_ref[...] + g_ref[...])


def _row_spec(width):
    return pl.BlockSpec((_BLK, width), lambda i: (i, 0))


def _full_spec(shape):
    return pl.BlockSpec(shape, lambda i: (0, 0))


def _layer0(X, W, b, deg):
    return pl.pallas_call(
        _l0_body,
        grid=(_N // _BLK,),
        in_specs=[
            _row_spec(_D),
            _full_spec(W.shape),
            _full_spec((1, W.shape[1])),
            _row_spec(1),
        ],
        out_specs=_row_spec(W.shape[1]),
        out_shape=jax.ShapeDtypeStruct((_N, W.shape[1]), jnp.float32),
    )(X, W, b.reshape(1, -1), deg)


def _layer_mid(S, G, deg, W, b):
    return pl.pallas_call(
        _mid_body,
        grid=(_N // _BLK,),
        in_specs=[
            _row_spec(_D),
            _row_spec(_D),
            _row_spec(1),
            _full_spec(W.shape),
            _full_spec((1, W.shape[1])),
        ],
        out_specs=_row_spec(W.shape[1]),
        out_shape=jax.ShapeDtypeStruct((_N, W.shape[1]), jnp.float32),
    )(S, G, deg, W, b.reshape(1, -1))


def _layer_fin(S, G, deg):
    width = G.shape[1]
    return pl.pallas_call(
        _fin_body,
        grid=(_N // _BLK,),
        in_specs=[_row_spec(width), _row_spec(width), _row_spec(1)],
        out_specs=_row_spec(width),
        out_shape=jax.ShapeDtypeStruct((_N, width), jnp.float32),
    )(S, G, deg)


def kernel(X, hyperedges, W0, b0, W1, b1, W2, b2):
    he = hyperedges.astype(jnp.int32)
    E, K = he.shape

    Xe = X[he]
    sq = jnp.sum(Xe * Xe, axis=-1)
    gram = jnp.einsum('ekd,emd->ekm', Xe, Xe)
    dist = sq[:, :, None] + sq[:, None, :] - 2.0 * gram
    flat = jnp.argmax(dist.reshape(E, K * K), axis=1)
    i = flat // K
    j = flat % K
    ar = jnp.arange(E)
    u = he[ar, i]
    v = he[ar, j]
    src = jnp.concatenate([u, v])
    dst = jnp.concatenate([v, u])

    npad = _EP - src.shape[0]
    src_p = jnp.concatenate([src, jnp.zeros((npad,), jnp.int32)])
    dst_p = jnp.concatenate([dst, jnp.full((npad,), _SENT, jnp.int32)])

    bins, cnts, degp = _bin(src_p, dst_p)
    deg = (degp[:_N] + degp[_DEGW:_DEGW + _N] + 1.0).reshape(_N, 1)

    W2p = jnp.pad(W2, ((0, 0), (0, 88)))
    b2p = jnp.pad(b2, (0, 88))

    G0 = _layer0(X, W0, b0, deg)
    S0 = _smooth(G0, bins, cnts)[:_N]
    G1 = _layer_mid(S0, G0, deg, W1, b1)
    S1 = _smooth(G1, bins, cnts)[:_N]
    G2 = _layer_mid(S1, G1, deg, W2p, b2p)
    S2 = _smooth(G2, bins, cnts)[:_N]
    return _layer_fin(S2, G2, deg)[:, :40]
